# baseline (speedup 1.0000x reference)
import functools

import jax
import jax.numpy as jnp
from jax import lax
from jax.experimental import pallas as pl
from jax.experimental.pallas import tpu as pltpu
from jax.experimental.pallas import tpu_sc as plsc

_NUM_WORKERS = 32
_CHUNK = 128


def _gather_body(n_chunks, b_per_w, uidx_hbm, iidx_hbm, uemb_hbm, iemb_hbm,
                 out_u, out_i, uidx_v, iidx_v, urows_v, irows_v, sem):
  wid = lax.axis_index("s") * 2 + lax.axis_index("c")
  base = wid * b_per_w
  pltpu.sync_copy(uidx_hbm.at[wid], uidx_v)
  pltpu.sync_copy(iidx_hbm.at[wid], iidx_v)
  copies = []
  for j in range(n_chunks):
    copies.append(pltpu.async_copy(
        uemb_hbm.at[uidx_v.at[j]], urows_v.at[pl.ds(j * _CHUNK, _CHUNK)], sem))
    copies.append(pltpu.async_copy(
        iemb_hbm.at[iidx_v.at[j]], irows_v.at[pl.ds(j * _CHUNK, _CHUNK)], sem))
  for c in copies:
    c.wait()
  pltpu.sync_copy(urows_v, out_u.at[pl.ds(base, b_per_w)])
  pltpu.sync_copy(irows_v, out_i.at[pl.ds(base, b_per_w)])


def _mlp_body(u_ref, i_ref, w1_ref, b1_ref, w2_ref, b2_ref, w3_ref, b3_ref,
              w4_ref, b4_ref, o_ref):
  nt = (((1,), (1,)), ((), ()))
  u = u_ref[...]
  i = i_ref[...]
  w1 = w1_ref[...]
  h = lax.dot_general(u, w1[:, :32], nt) + lax.dot_general(i, w1[:, 32:], nt)
  h = jnp.maximum(h + b1_ref[...], 0.0)
  h = jnp.maximum(lax.dot_general(h, w2_ref[...], nt) + b2_ref[...], 0.0)
  h = jnp.maximum(lax.dot_general(h, w3_ref[...], nt) + b3_ref[...], 0.0)
  z = jnp.sum(h * w4_ref[...], axis=1, keepdims=True) + b4_ref[...]
  o_ref[...] = jax.nn.sigmoid(z)


def kernel(user_indices, item_indices, user_emb, item_emb,
           W1, b1, W2, b2, W3, b3, W4, b4):
  batch = user_indices.shape[0]
  emb_dim = user_emb.shape[1]
  b_per_w = batch // _NUM_WORKERS
  n_chunks = b_per_w // _CHUNK

  mesh = plsc.VectorSubcoreMesh(core_axis_name="c", subcore_axis_name="s")
  gather = functools.partial(
      pl.kernel,
      out_type=[jax.ShapeDtypeStruct((batch, emb_dim), jnp.float32),
                jax.ShapeDtypeStruct((batch, emb_dim), jnp.float32)],
      mesh=mesh,
      scratch_types=[
          pltpu.VMEM((n_chunks, _CHUNK), jnp.int32),
          pltpu.VMEM((n_chunks, _CHUNK), jnp.int32),
          pltpu.VMEM((b_per_w, emb_dim), jnp.float32),
          pltpu.VMEM((b_per_w, emb_dim), jnp.float32),
          pltpu.SemaphoreType.DMA,
      ],
      compiler_params=pltpu.CompilerParams(use_tc_tiling_on_sc=False),
  )(functools.partial(_gather_body, n_chunks, b_per_w))

  u_rows, i_rows = gather(
      user_indices.reshape(_NUM_WORKERS, n_chunks, _CHUNK),
      item_indices.reshape(_NUM_WORKERS, n_chunks, _CHUNK),
      user_emb, item_emb)

  blk = 2048
  grid = (batch // blk,)
  full = lambda shape: pl.BlockSpec(shape, lambda j: (0, 0))
  predict = pl.pallas_call(
      _mlp_body,
      grid=grid,
      in_specs=[
          pl.BlockSpec((blk, emb_dim), lambda j: (j, 0)),
          pl.BlockSpec((blk, emb_dim), lambda j: (j, 0)),
          full(W1.shape),
          full((1, b1.shape[0])),
          full(W2.shape),
          full((1, b2.shape[0])),
          full(W3.shape),
          full((1, b3.shape[0])),
          full(W4.shape),
          full((1, 1)),
      ],
      out_specs=pl.BlockSpec((blk, 1), lambda j: (j, 0)),
      out_shape=jax.ShapeDtypeStruct((batch, 1), jnp.float32),
  )(u_rows, i_rows, W1, b1.reshape(1, -1), W2, b2.reshape(1, -1),
    W3, b3.reshape(1, -1), W4, b4.reshape(1, 1))
  return predict
